# Initial kernel scaffold; baseline (speedup 1.0000x reference)
import jax
import jax.numpy as jnp
from jax.experimental import pallas as pl


def kernel(features, edge_index, W1, al1, ar1, b1, W2, al2, ar2, b2, W3, al3, ar3, b3):
    raise NotImplementedError("write your pallas kernel here")


# speedup vs baseline: 39.2507x; 39.2507x over previous
import functools

import jax
import jax.numpy as jnp
from jax import lax
from jax.experimental import pallas as pl
from jax.experimental.pallas import tpu as pltpu
from jax.experimental.pallas import tpu_sc as plsc

N = 10000
E = 320000
IN = 128
NP = 10112
NC = 2
NS = 16
NW = NC * NS
K = 128
RPS = NP // NS


def _make_edge_kernel(C, D):
    G = C // 16
    NCHUNK = E // K
    EXTRA = NCHUNK % NW
    mesh = plsc.VectorSubcoreMesh(core_axis_name="c", subcore_axis_name="s")

    @functools.partial(
        pl.kernel,
        out_type=[jax.ShapeDtypeStruct((NC, NP, C), jnp.float32),
                  jax.ShapeDtypeStruct((NC, NP, 16), jnp.float32)],
        mesh=mesh,
        compiler_params=pltpu.CompilerParams(use_tc_tiling_on_sc=False),
        scratch_types=[
            pltpu.VMEM_SHARED((NP, C), jnp.float32),
            pltpu.VMEM_SHARED((NP, 16), jnp.float32),
            pltpu.VMEM((K,), jnp.int32),
            pltpu.VMEM((K,), jnp.int32),
            pltpu.VMEM((K, C), jnp.float32),
            pltpu.VMEM((K, 16), jnp.float32),
            pltpu.VMEM((K, 16), jnp.float32),
            pltpu.VMEM((K, C), jnp.float32),
            pltpu.VMEM((K, 16), jnp.float32),
            pltpu.SemaphoreType.DMA,
            pltpu.SemaphoreType.DMA,
            pltpu.SemaphoreType.DMA,
        ],
    )
    def ek(t_hbm, l_hbm, r_hbm, src_hbm, dst_hbm, out_h, out_e,
           acc_h, acc_e, idx_s, idx_d, rows, lrows, rrows, msg_h, msg_e,
           sem_t, sem_l, sem_r):
        cid = lax.axis_index("c")
        sid = lax.axis_index("s")
        w = sid * NC + cid

        def zb(k, carry):
            for g in range(G):
                msg_h[k, pl.ds(g * 16, 16)] = jnp.zeros((16,), jnp.float32)
            msg_e[k, :] = jnp.zeros((16,), jnp.float32)
            return carry
        lax.fori_loop(0, K, zb, 0)
        for i in range(RPS // K):
            pltpu.sync_copy(msg_h, acc_h.at[pl.ds(sid * RPS + i * K, K)])
            pltpu.sync_copy(msg_e, acc_e.at[pl.ds(sid * RPS + i * K, K)])
        if RPS % K:
            rem = RPS % K
            base = sid * RPS + (RPS // K) * K
            pltpu.sync_copy(msg_h.at[pl.ds(0, rem)],
                            acc_h.at[pl.ds(base, rem)])
            pltpu.sync_copy(msg_e.at[pl.ds(0, rem)],
                            acc_e.at[pl.ds(base, rem)])
        plsc.subcore_barrier()

        nchunks_w = NCHUNK // NW + jnp.where(w < EXTRA, 1, 0)

        def chunk_body(j, carry):
            off = (w + j * NW) * K
            pltpu.sync_copy(src_hbm.at[pl.ds(off, K)], idx_s)
            pltpu.sync_copy(dst_hbm.at[pl.ds(off, K)], idx_d)
            ct = pltpu.async_copy(t_hbm.at[idx_s], rows, sem_t)
            cl = pltpu.async_copy(l_hbm.at[idx_s], lrows, sem_l)
            cr = pltpu.async_copy(r_hbm.at[idx_d], rrows, sem_r)
            ct.wait()
            cl.wait()
            cr.wait()

            def edge_body(k, ecarry):
                e = lrows[k, :] + rrows[k, :]
                e = jnp.where(e > 0.0, e, 0.2 * e)
                ee = jnp.exp(e)
                msg_e[k, :] = ee
                for g in range(G):
                    s = ee[(g * 16) // D]
                    msg_h[k, pl.ds(g * 16, 16)] = (
                        rows[k, pl.ds(g * 16, 16)] * s)
                return ecarry
            lax.fori_loop(0, K, edge_body, 0)

            pltpu.sync_copy(msg_h, acc_h.at[idx_d], add=True)
            pltpu.sync_copy(msg_e, acc_e.at[idx_d], add=True)
            return carry
        lax.fori_loop(0, nchunks_w, chunk_body, 0)

        plsc.subcore_barrier()
        pltpu.sync_copy(acc_h.at[pl.ds(sid * RPS, RPS)],
                        out_h.at[cid, pl.ds(sid * RPS, RPS)])
        pltpu.sync_copy(acc_e.at[pl.ds(sid * RPS, RPS)],
                        out_e.at[cid, pl.ds(sid * RPS, RPS)])

    return ek


_ek1 = _make_edge_kernel(128, 16)
_ek2 = _make_edge_kernel(16, 16)
_ek3 = _make_edge_kernel(32, 32)


NB = 1264
_GRID = NP // NB


def _rows(width):
    return pl.BlockSpec((NB, width), lambda i: (i, 0))


def _part(width):
    return pl.BlockSpec((NC, NB, width), lambda i: (0, i, 0))


def _full(shape):
    return pl.BlockSpec(shape, lambda i: tuple(0 for _ in shape))


def _proj_body(x_ref, w_ref, wl_ref, wr_ref, t_ref, l_ref, r_ref):
    x = x_ref[...]
    t_ref[...] = jnp.dot(x, w_ref[...], preferred_element_type=jnp.float32)
    l_ref[...] = jnp.dot(x, wl_ref[...], preferred_element_type=jnp.float32)
    r_ref[...] = jnp.dot(x, wr_ref[...], preferred_element_type=jnp.float32)


def _proj(x, w, wl, wr):
    return pl.pallas_call(
        _proj_body,
        grid=(_GRID,),
        in_specs=[_rows(IN), _full(w.shape), _full(wl.shape), _full(wr.shape)],
        out_specs=[_rows(w.shape[1]), _rows(16), _rows(16)],
        out_shape=[
            jax.ShapeDtypeStruct((x.shape[0], w.shape[1]), jnp.float32),
            jax.ShapeDtypeStruct((x.shape[0], 16), jnp.float32),
            jax.ShapeDtypeStruct((x.shape[0], 16), jnp.float32),
        ],
    )(x, w, wl, wr)


def _finalize(p_h, p_e, heads, d, b):
    acc = p_h[0] + p_h[1]
    den = p_e[0] + p_e[1]
    parts = []
    for h in range(heads):
        num = acc[:, d * h:d * (h + 1)]
        dh = jnp.maximum(den[:, h:h + 1], 1e-9)
        parts.append(num / dh)
    out = parts[0] if len(parts) == 1 else jnp.concatenate(parts, axis=1)
    return out + b


def _comb1_body(ph_ref, pe_ref, b_ref, w_ref, wl_ref, wr_ref,
                t_ref, l_ref, r_ref):
    h = _finalize(ph_ref[...], pe_ref[...], 8, 16, b_ref[...])
    h = jnp.maximum(h, 0.0)
    t_ref[...] = jnp.dot(h, w_ref[...], preferred_element_type=jnp.float32)
    l_ref[...] = jnp.dot(h, wl_ref[...], preferred_element_type=jnp.float32)
    r_ref[...] = jnp.dot(h, wr_ref[...], preferred_element_type=jnp.float32)


def _comb2_body(ph_ref, pe_ref, b_ref, w_ref, wl_ref, wr_ref,
                emb_ref, t_ref, l_ref, r_ref):
    h = _finalize(ph_ref[...], pe_ref[...], 1, 16, b_ref[...])
    emb_ref[...] = h
    t_ref[...] = jnp.dot(h, w_ref[...], preferred_element_type=jnp.float32)
    l_ref[...] = jnp.dot(h, wl_ref[...], preferred_element_type=jnp.float32)
    r_ref[...] = jnp.dot(h, wr_ref[...], preferred_element_type=jnp.float32)


def _comb3_body(ph_ref, pe_ref, b_ref, out_ref):
    out_ref[...] = _finalize(ph_ref[...], pe_ref[...], 1, 32, b_ref[...])


def _blockdiag_att(a):
    heads, d = a.shape
    m = jnp.zeros((heads * d, heads), dtype=a.dtype)
    for h in range(heads):
        m = m.at[h * d:(h + 1) * d, h].set(a[h])
    return m


def kernel(features, edge_index, W1, al1, ar1, b1, W2, al2, ar2, b2,
           W3, al3, ar3, b3):
    src = edge_index[0]
    dst = edge_index[1]

    def aug(W, al, ar, heads, d):
        albd = _blockdiag_att(al.reshape(heads, d))
        arbd = _blockdiag_att(ar.reshape(heads, d))
        zt = jnp.zeros((W.shape[0], 16 - heads), jnp.float32)
        wl = jnp.concatenate([W @ albd, zt], axis=1)
        wr = jnp.concatenate([W @ arbd, zt], axis=1)
        return wl, wr

    wl1, wr1 = aug(W1, al1, ar1, 8, 16)
    wl2, wr2 = aug(W2, al2, ar2, 1, 16)
    wl3, wr3 = aug(W3, al3, ar3, 1, 32)

    xp = jnp.zeros((NP, IN), jnp.float32).at[:N].set(features)

    t1, l1, r1 = _proj(xp, W1, wl1, wr1)
    p1h, p1e = _ek1(t1, l1, r1, src, dst)

    t2, l2, r2 = pl.pallas_call(
        _comb1_body,
        grid=(_GRID,),
        in_specs=[_part(128), _part(16), _full((1, 128)),
                  _full((128, 16)), _full((128, 16)), _full((128, 16))],
        out_specs=[_rows(16), _rows(16), _rows(16)],
        out_shape=[jax.ShapeDtypeStruct((NP, 16), jnp.float32),
                   jax.ShapeDtypeStruct((NP, 16), jnp.float32),
                   jax.ShapeDtypeStruct((NP, 16), jnp.float32)],
    )(p1h, p1e, b1.reshape(1, 128), W2, wl2, wr2)
    p2h, p2e = _ek2(t2, l2, r2, src, dst)

    emb, t3, l3, r3 = pl.pallas_call(
        _comb2_body,
        grid=(_GRID,),
        in_specs=[_part(16), _part(16), _full((1, 16)),
                  _full((16, 32)), _full((16, 16)), _full((16, 16))],
        out_specs=[_rows(16), _rows(32), _rows(16), _rows(16)],
        out_shape=[jax.ShapeDtypeStruct((NP, 16), jnp.float32),
                   jax.ShapeDtypeStruct((NP, 32), jnp.float32),
                   jax.ShapeDtypeStruct((NP, 16), jnp.float32),
                   jax.ShapeDtypeStruct((NP, 16), jnp.float32)],
    )(p2h, p2e, b2.reshape(1, 16), W3, wl3, wr3)
    p3h, p3e = _ek3(t3, l3, r3, src, dst)

    out = pl.pallas_call(
        _comb3_body,
        grid=(_GRID,),
        in_specs=[_part(32), _part(16), _full((1, 32))],
        out_specs=_rows(32),
        out_shape=jax.ShapeDtypeStruct((NP, 32), jnp.float32),
    )(p3h, p3e, b3.reshape(1, 32))

    return out[:N], emb[:N]


# speedup vs baseline: 39.8264x; 1.0147x over previous
import functools

import jax
import jax.numpy as jnp
from jax import lax
from jax.experimental import pallas as pl
from jax.experimental.pallas import tpu as pltpu
from jax.experimental.pallas import tpu_sc as plsc

N = 10000
E = 320000
IN = 128
NP = 10112
NC = 2
NS = 16
NW = NC * NS
K = 64
RPS = NP // NS


CHW = 158
IDXR = NW * CHW + 2


def _make_edge_kernel(C, D):
    G = C // 16
    mesh = plsc.VectorSubcoreMesh(core_axis_name="c", subcore_axis_name="s")

    @functools.partial(
        pl.kernel,
        out_type=[jax.ShapeDtypeStruct((NC, NP, C), jnp.float32),
                  jax.ShapeDtypeStruct((NC, NP, 16), jnp.float32)],
        mesh=mesh,
        compiler_params=pltpu.CompilerParams(use_tc_tiling_on_sc=False),
        scratch_types=[
            pltpu.VMEM_SHARED((NP, C), jnp.float32),
            pltpu.VMEM_SHARED((NP, 16), jnp.float32),
            pltpu.VMEM((K,), jnp.int32),
            pltpu.VMEM((K,), jnp.int32),
            pltpu.VMEM((K,), jnp.int32),
            pltpu.VMEM((K,), jnp.int32),
            pltpu.VMEM((K, C), jnp.float32),
            pltpu.VMEM((K, C), jnp.float32),
            pltpu.VMEM((K, 16), jnp.float32),
            pltpu.VMEM((K, 16), jnp.float32),
            pltpu.VMEM((K, 16), jnp.float32),
            pltpu.VMEM((K, 16), jnp.float32),
            pltpu.VMEM((K, C), jnp.float32),
            pltpu.VMEM((K, 16), jnp.float32),
            pltpu.SemaphoreType.DMA,
            pltpu.SemaphoreType.DMA,
        ],
    )
    def ek(t_hbm, l_hbm, r_hbm, srcm, dstm, out_h, out_e,
           acc_h, acc_e, idxs0, idxs1, idxd0, idxd1,
           rows0, rows1, lrows0, lrows1, rrows0, rrows1,
           msgh, msge, gsem0, gsem1):
        cid = lax.axis_index("c")
        sid = lax.axis_index("s")
        w = sid * NC + cid
        rows = (rows0, rows1)
        lrows = (lrows0, lrows1)
        rrows = (rrows0, rrows1)
        idxs = (idxs0, idxs1)
        idxd = (idxd0, idxd1)
        gsem = (gsem0, gsem1)
        base = w * CHW

        def zb(k, carry):
            for g in range(G):
                msgh[k, pl.ds(g * 16, 16)] = jnp.zeros((16,), jnp.float32)
            msge[k, :] = jnp.zeros((16,), jnp.float32)
            return carry
        lax.fori_loop(0, K, zb, 0)
        for i in range(RPS // K):
            pltpu.sync_copy(msgh, acc_h.at[pl.ds(sid * RPS + i * K, K)])
            pltpu.sync_copy(msge, acc_e.at[pl.ds(sid * RPS + i * K, K)])
        if RPS % K:
            rem = RPS % K
            zbase = sid * RPS + (RPS // K) * K
            pltpu.sync_copy(msgh.at[pl.ds(0, rem)],
                            acc_h.at[pl.ds(zbase, rem)])
            pltpu.sync_copy(msge.at[pl.ds(0, rem)],
                            acc_e.at[pl.ds(zbase, rem)])
        plsc.subcore_barrier()

        def issue_gathers(j, b):
            pltpu.sync_copy(srcm.at[base + j], idxs[b])
            pltpu.sync_copy(dstm.at[base + j], idxd[b])
            pltpu.async_copy(t_hbm.at[idxs[b]], rows[b], gsem[b])
            pltpu.async_copy(l_hbm.at[idxs[b]], lrows[b], gsem[b])
            pltpu.async_copy(r_hbm.at[idxd[b]], rrows[b], gsem[b])

        def wait_gathers(b):
            pltpu.make_async_copy(t_hbm.at[idxs[b]], rows[b],
                                  gsem[b]).wait()
            pltpu.make_async_copy(l_hbm.at[idxs[b]], lrows[b],
                                  gsem[b]).wait()
            pltpu.make_async_copy(r_hbm.at[idxd[b]], rrows[b],
                                  gsem[b]).wait()

        def do_scatters(b):
            pltpu.sync_copy(msgh, acc_h.at[idxd[b]], add=True)
            pltpu.sync_copy(msge, acc_e.at[idxd[b]], add=True)

        def compute(b):
            lr, rr, ro, mh, me = lrows[b], rrows[b], rows[b], msgh, msge

            def edge_body(k, ecarry):
                e = lr[k, :] + rr[k, :]
                e = jnp.where(e > 0.0, e, 0.2 * e)
                ee = jnp.exp(e)
                me[k, :] = ee
                for g in range(G):
                    s = ee[(g * 16) // D]
                    mh[k, pl.ds(g * 16, 16)] = ro[k, pl.ds(g * 16, 16)] * s
                return ecarry
            lax.fori_loop(0, K, edge_body, 0)

        issue_gathers(0, 0)
        issue_gathers(1, 1)

        def loop_body(i, carry):
            for b in (0, 1):
                j = 2 * i + b
                wait_gathers(b)
                compute(b)
                do_scatters(b)
                issue_gathers(j + 2, b)
            return carry
        lax.fori_loop(0, CHW // 2, loop_body, 0)

        wait_gathers(0)
        wait_gathers(1)

        plsc.subcore_barrier()
        pltpu.sync_copy(acc_h.at[pl.ds(sid * RPS, RPS)],
                        out_h.at[cid, pl.ds(sid * RPS, RPS)])
        pltpu.sync_copy(acc_e.at[pl.ds(sid * RPS, RPS)],
                        out_e.at[cid, pl.ds(sid * RPS, RPS)])

    return ek


_ek1 = _make_edge_kernel(128, 16)
_ek2 = _make_edge_kernel(16, 16)
_ek3 = _make_edge_kernel(32, 32)


NB = 1264
_GRID = NP // NB


def _rows(width):
    return pl.BlockSpec((NB, width), lambda i: (i, 0))


def _part(width):
    return pl.BlockSpec((NC, NB, width), lambda i: (0, i, 0))


def _full(shape):
    return pl.BlockSpec(shape, lambda i: tuple(0 for _ in shape))


def _proj_body(x_ref, w_ref, wl_ref, wr_ref, t_ref, l_ref, r_ref):
    x = x_ref[...]
    t_ref[...] = jnp.dot(x, w_ref[...], preferred_element_type=jnp.float32)
    l_ref[...] = jnp.dot(x, wl_ref[...], preferred_element_type=jnp.float32)
    r_ref[...] = jnp.dot(x, wr_ref[...], preferred_element_type=jnp.float32)


def _proj(x, w, wl, wr):
    return pl.pallas_call(
        _proj_body,
        grid=(_GRID,),
        in_specs=[_rows(IN), _full(w.shape), _full(wl.shape), _full(wr.shape)],
        out_specs=[_rows(w.shape[1]), _rows(16), _rows(16)],
        out_shape=[
            jax.ShapeDtypeStruct((x.shape[0], w.shape[1]), jnp.float32),
            jax.ShapeDtypeStruct((x.shape[0], 16), jnp.float32),
            jax.ShapeDtypeStruct((x.shape[0], 16), jnp.float32),
        ],
    )(x, w, wl, wr)


def _finalize(p_h, p_e, heads, d, b):
    acc = p_h[0] + p_h[1]
    den = p_e[0] + p_e[1]
    parts = []
    for h in range(heads):
        num = acc[:, d * h:d * (h + 1)]
        dh = jnp.maximum(den[:, h:h + 1], 1e-9)
        parts.append(num / dh)
    out = parts[0] if len(parts) == 1 else jnp.concatenate(parts, axis=1)
    return out + b


def _comb1_body(ph_ref, pe_ref, b_ref, w_ref, wl_ref, wr_ref,
                t_ref, l_ref, r_ref):
    h = _finalize(ph_ref[...], pe_ref[...], 8, 16, b_ref[...])
    h = jnp.maximum(h, 0.0)
    t_ref[...] = jnp.dot(h, w_ref[...], preferred_element_type=jnp.float32)
    l_ref[...] = jnp.dot(h, wl_ref[...], preferred_element_type=jnp.float32)
    r_ref[...] = jnp.dot(h, wr_ref[...], preferred_element_type=jnp.float32)


def _comb2_body(ph_ref, pe_ref, b_ref, w_ref, wl_ref, wr_ref,
                emb_ref, t_ref, l_ref, r_ref):
    h = _finalize(ph_ref[...], pe_ref[...], 1, 16, b_ref[...])
    emb_ref[...] = h
    t_ref[...] = jnp.dot(h, w_ref[...], preferred_element_type=jnp.float32)
    l_ref[...] = jnp.dot(h, wl_ref[...], preferred_element_type=jnp.float32)
    r_ref[...] = jnp.dot(h, wr_ref[...], preferred_element_type=jnp.float32)


def _comb3_body(ph_ref, pe_ref, b_ref, out_ref):
    out_ref[...] = _finalize(ph_ref[...], pe_ref[...], 1, 32, b_ref[...])


def _blockdiag_att(a):
    heads, d = a.shape
    m = jnp.zeros((heads * d, heads), dtype=a.dtype)
    for h in range(heads):
        m = m.at[h * d:(h + 1) * d, h].set(a[h])
    return m


def kernel(features, edge_index, W1, al1, ar1, b1, W2, al2, ar2, b2,
           W3, al3, ar3, b3):
    src = jnp.full((IDXR * K,), N, jnp.int32).at[:E].set(
        edge_index[0]).reshape(IDXR, K)
    dst = jnp.full((IDXR * K,), N, jnp.int32).at[:E].set(
        edge_index[1]).reshape(IDXR, K)

    def aug(W, al, ar, heads, d):
        albd = _blockdiag_att(al.reshape(heads, d))
        arbd = _blockdiag_att(ar.reshape(heads, d))
        zt = jnp.zeros((W.shape[0], 16 - heads), jnp.float32)
        wl = jnp.concatenate([W @ albd, zt], axis=1)
        wr = jnp.concatenate([W @ arbd, zt], axis=1)
        return wl, wr

    wl1, wr1 = aug(W1, al1, ar1, 8, 16)
    wl2, wr2 = aug(W2, al2, ar2, 1, 16)
    wl3, wr3 = aug(W3, al3, ar3, 1, 32)

    xp = jnp.zeros((NP, IN), jnp.float32).at[:N].set(features)

    t1, l1, r1 = _proj(xp, W1, wl1, wr1)
    p1h, p1e = _ek1(t1, l1, r1, src, dst)

    t2, l2, r2 = pl.pallas_call(
        _comb1_body,
        grid=(_GRID,),
        in_specs=[_part(128), _part(16), _full((1, 128)),
                  _full((128, 16)), _full((128, 16)), _full((128, 16))],
        out_specs=[_rows(16), _rows(16), _rows(16)],
        out_shape=[jax.ShapeDtypeStruct((NP, 16), jnp.float32),
                   jax.ShapeDtypeStruct((NP, 16), jnp.float32),
                   jax.ShapeDtypeStruct((NP, 16), jnp.float32)],
    )(p1h, p1e, b1.reshape(1, 128), W2, wl2, wr2)
    p2h, p2e = _ek2(t2, l2, r2, src, dst)

    emb, t3, l3, r3 = pl.pallas_call(
        _comb2_body,
        grid=(_GRID,),
        in_specs=[_part(16), _part(16), _full((1, 16)),
                  _full((16, 32)), _full((16, 16)), _full((16, 16))],
        out_specs=[_rows(16), _rows(32), _rows(16), _rows(16)],
        out_shape=[jax.ShapeDtypeStruct((NP, 16), jnp.float32),
                   jax.ShapeDtypeStruct((NP, 32), jnp.float32),
                   jax.ShapeDtypeStruct((NP, 16), jnp.float32),
                   jax.ShapeDtypeStruct((NP, 16), jnp.float32)],
    )(p2h, p2e, b2.reshape(1, 16), W3, wl3, wr3)
    p3h, p3e = _ek3(t3, l3, r3, src, dst)

    out = pl.pallas_call(
        _comb3_body,
        grid=(_GRID,),
        in_specs=[_part(32), _part(16), _full((1, 32))],
        out_specs=_rows(32),
        out_shape=jax.ShapeDtypeStruct((NP, 32), jnp.float32),
    )(p3h, p3e, b3.reshape(1, 32))

    return out[:N], emb[:N]


# speedup vs baseline: 71.3501x; 1.7915x over previous
import functools

import jax
import jax.numpy as jnp
from jax import lax
from jax.experimental import pallas as pl
from jax.experimental.pallas import tpu as pltpu
from jax.experimental.pallas import tpu_sc as plsc

N = 10000
E = 320000
IN = 128
NP = 10112
NC = 2
NS = 16
NW = NC * NS
K = 80
RPS = NP // NS
CHW = 126
IDXR = NW * CHW + 2


def _make_edge_kernel(C, D):
    CW = C + 16
    G = C // 16
    HEADS = C // D
    mesh = plsc.VectorSubcoreMesh(core_axis_name="c", subcore_axis_name="s")

    @functools.partial(
        pl.kernel,
        out_type=jax.ShapeDtypeStruct((NC, NP, CW), jnp.float32),
        mesh=mesh,
        compiler_params=pltpu.CompilerParams(use_tc_tiling_on_sc=False,
                                             needs_layout_passes=False),
        scratch_types=[
            pltpu.VMEM_SHARED((NP, CW), jnp.float32),
            pltpu.VMEM((2, K), jnp.int32),
            pltpu.VMEM((2, K), jnp.int32),
            pltpu.VMEM((K, CW), jnp.float32),
            pltpu.VMEM((K, CW), jnp.float32),
            pltpu.VMEM((K, 16), jnp.float32),
            pltpu.VMEM((K, 16), jnp.float32),
            pltpu.VMEM((K, CW), jnp.float32),
            pltpu.SemaphoreType.DMA,
            pltpu.SemaphoreType.DMA,
        ],
    )
    def ek(t_hbm, r_hbm, sdm, out_hbm,
           acc, idx0, idx1, rows0, rows1, rrows0, rrows1, msg,
           gsem0, gsem1):
        cid = lax.axis_index("c")
        sid = lax.axis_index("s")
        w = sid * NC + cid
        rows = (rows0, rows1)
        rrows = (rrows0, rrows1)
        idx = (idx0, idx1)
        gsem = (gsem0, gsem1)
        base = w * CHW

        def zb(k, carry):
            for g in range(G + 1):
                msg[k, pl.ds(g * 16, 16)] = jnp.zeros((16,), jnp.float32)
            return carry
        lax.fori_loop(0, K, zb, 0)
        for i in range(RPS // K):
            pltpu.sync_copy(msg, acc.at[pl.ds(sid * RPS + i * K, K)])
        if RPS % K:
            rem = RPS % K
            zbase = sid * RPS + (RPS // K) * K
            pltpu.sync_copy(msg.at[pl.ds(0, rem)],
                            acc.at[pl.ds(zbase, rem)])
        plsc.subcore_barrier()

        def issue_gathers(j, b):
            pltpu.sync_copy(sdm.at[base + j], idx[b])
            pltpu.async_copy(t_hbm.at[idx[b].at[0]], rows[b], gsem[b])
            pltpu.async_copy(r_hbm.at[idx[b].at[1]], rrows[b], gsem[b])

        def wait_gathers(b):
            pltpu.make_async_copy(t_hbm.at[idx[b].at[0]], rows[b],
                                  gsem[b]).wait()
            pltpu.make_async_copy(r_hbm.at[idx[b].at[1]], rrows[b],
                                  gsem[b]).wait()

        def compute(b):
            ro, rr = rows[b], rrows[b]
            if HEADS > 1:
                def edge_body(k, ecarry):
                    e = ro[k, pl.ds(C, 16)] + rr[k, :]
                    e = jnp.where(e > 0.0, e, 0.2 * e)
                    ee = jnp.exp(e)
                    msg[k, pl.ds(C, 16)] = ee
                    for g in range(G):
                        s = ee[(g * 16) // D]
                        msg[k, pl.ds(g * 16, 16)] = (
                            ro[k, pl.ds(g * 16, 16)] * s)
                    return ecarry
                lax.fori_loop(0, K, edge_body, 0)
            else:
                lanes = jnp.arange(16, dtype=jnp.int32)
                cC = jnp.full((16,), C, jnp.int32)
                c0 = jnp.zeros((16,), jnp.int32)
                for kb in range(K // 16):
                    kvec = kb * 16 + lanes
                    el16 = plsc.load_gather(ro, [kvec, cC])
                    er16 = plsc.load_gather(rr, [kvec, c0])
                    e = el16 + er16
                    e = jnp.where(e > 0.0, e, 0.2 * e)
                    ee16 = jnp.exp(e)
                    plsc.store_scatter(msg, [kvec, cC], ee16)
                    for k2 in range(16):
                        k = kb * 16 + k2
                        s = ee16[k2]
                        for g in range(G):
                            msg[k, pl.ds(g * 16, 16)] = (
                                ro[k, pl.ds(g * 16, 16)] * s)

        issue_gathers(0, 0)
        issue_gathers(1, 1)

        def loop_body(i, carry):
            for b in (0, 1):
                j = 2 * i + b
                wait_gathers(b)
                compute(b)
                pltpu.sync_copy(msg, acc.at[idx[b].at[1]], add=True)
                issue_gathers(j + 2, b)
            return carry
        lax.fori_loop(0, CHW // 2, loop_body, 0)

        wait_gathers(0)
        wait_gathers(1)

        plsc.subcore_barrier()
        pltpu.sync_copy(acc.at[pl.ds(sid * RPS, RPS)],
                        out_hbm.at[cid, pl.ds(sid * RPS, RPS)])

    return ek


_ek1 = _make_edge_kernel(128, 16)
_ek2 = _make_edge_kernel(16, 16)
_ek3 = _make_edge_kernel(32, 32)


NB = 1264
_GRID = NP // NB


def _rows(width):
    return pl.BlockSpec((NB, width), lambda i: (i, 0))


def _part(width):
    return pl.BlockSpec((NC, NB, width), lambda i: (0, i, 0))


def _full(shape):
    return pl.BlockSpec(shape, lambda i: tuple(0 for _ in shape))


def _proj_body(x_ref, wt_ref, wr_ref, t_ref, r_ref):
    x = x_ref[...]
    t_ref[...] = jnp.dot(x, wt_ref[...], preferred_element_type=jnp.float32)
    r_ref[...] = jnp.dot(x, wr_ref[...], preferred_element_type=jnp.float32)


def _proj(x, wt, wr):
    return pl.pallas_call(
        _proj_body,
        grid=(_GRID,),
        in_specs=[_rows(x.shape[1]), _full(wt.shape), _full(wr.shape)],
        out_specs=[_rows(wt.shape[1]), _rows(16)],
        out_shape=[
            jax.ShapeDtypeStruct((x.shape[0], wt.shape[1]), jnp.float32),
            jax.ShapeDtypeStruct((x.shape[0], 16), jnp.float32),
        ],
    )(x, wt, wr)


def _finalize(p, heads, d, b):
    acc = p[0] + p[1]
    c = heads * d
    parts = []
    for h in range(heads):
        num = acc[:, d * h:d * (h + 1)]
        den = jnp.maximum(acc[:, c + h:c + h + 1], 1e-9)
        parts.append(num / den)
    out = parts[0] if len(parts) == 1 else jnp.concatenate(parts, axis=1)
    return out + b


def _comb1_body(p_ref, b_ref, wt_ref, wr_ref, t_ref, r_ref):
    h = _finalize(p_ref[...], 8, 16, b_ref[...])
    h = jnp.maximum(h, 0.0)
    t_ref[...] = jnp.dot(h, wt_ref[...], preferred_element_type=jnp.float32)
    r_ref[...] = jnp.dot(h, wr_ref[...], preferred_element_type=jnp.float32)


def _comb2_body(p_ref, b_ref, wt_ref, wr_ref, emb_ref, t_ref, r_ref):
    h = _finalize(p_ref[...], 1, 16, b_ref[...])
    emb_ref[...] = h
    t_ref[...] = jnp.dot(h, wt_ref[...], preferred_element_type=jnp.float32)
    r_ref[...] = jnp.dot(h, wr_ref[...], preferred_element_type=jnp.float32)


def _comb3_body(p_ref, b_ref, out_ref):
    out_ref[...] = _finalize(p_ref[...], 1, 32, b_ref[...])


def _blockdiag_att(a):
    heads, d = a.shape
    m = jnp.zeros((heads * d, heads), dtype=a.dtype)
    for h in range(heads):
        m = m.at[h * d:(h + 1) * d, h].set(a[h])
    return m


def kernel(features, edge_index, W1, al1, ar1, b1, W2, al2, ar2, b2,
           W3, al3, ar3, b3):
    src = jnp.full((IDXR * K,), N, jnp.int32).at[:E].set(
        edge_index[0]).reshape(IDXR, K)
    dst = jnp.full((IDXR * K,), N, jnp.int32).at[:E].set(
        edge_index[1]).reshape(IDXR, K)
    sdm = jnp.stack([src, dst], axis=1)

    def aug(W, al, ar, heads, d):
        albd = _blockdiag_att(al.reshape(heads, d))
        arbd = _blockdiag_att(ar.reshape(heads, d))
        zt = jnp.zeros((W.shape[0], 16 - heads), jnp.float32)
        wt = jnp.concatenate([W, W @ albd, zt], axis=1)
        wr = jnp.concatenate([W @ arbd, zt], axis=1)
        return wt, wr

    wt1, wr1 = aug(W1, al1, ar1, 8, 16)
    wt2, wr2 = aug(W2, al2, ar2, 1, 16)
    wt3, wr3 = aug(W3, al3, ar3, 1, 32)

    xp = jnp.zeros((NP, IN), jnp.float32).at[:N].set(features)

    t1, r1 = _proj(xp, wt1, wr1)
    p1 = _ek1(t1, r1, sdm)

    t2, r2 = pl.pallas_call(
        _comb1_body,
        grid=(_GRID,),
        in_specs=[_part(144), _full((1, 128)),
                  _full((128, 32)), _full((128, 16))],
        out_specs=[_rows(32), _rows(16)],
        out_shape=[jax.ShapeDtypeStruct((NP, 32), jnp.float32),
                   jax.ShapeDtypeStruct((NP, 16), jnp.float32)],
    )(p1, b1.reshape(1, 128), wt2, wr2)
    p2 = _ek2(t2, r2, sdm)

    emb, t3, r3 = pl.pallas_call(
        _comb2_body,
        grid=(_GRID,),
        in_specs=[_part(32), _full((1, 16)),
                  _full((16, 48)), _full((16, 16))],
        out_specs=[_rows(16), _rows(48), _rows(16)],
        out_shape=[jax.ShapeDtypeStruct((NP, 16), jnp.float32),
                   jax.ShapeDtypeStruct((NP, 48), jnp.float32),
                   jax.ShapeDtypeStruct((NP, 16), jnp.float32)],
    )(p2, b2.reshape(1, 16), wt3, wr3)
    p3 = _ek3(t3, r3, sdm)

    out = pl.pallas_call(
        _comb3_body,
        grid=(_GRID,),
        in_specs=[_part(48), _full((1, 32))],
        out_specs=_rows(32),
        out_shape=jax.ShapeDtypeStruct((NP, 32), jnp.float32),
    )(p3, b3.reshape(1, 32))

    return out[:N], emb[:N]
